# baseline (speedup 1.0000x reference)
import functools

import jax
import jax.numpy as jnp
from jax import lax
from jax.experimental import pallas as pl
from jax.experimental.pallas import tpu as pltpu
from jax.experimental.pallas import tpu_sc as plsc

_TOPK = 8
_ROWS = 256


def _topk_body(q_ref, k_ref, w_ref, i_ref):
    q = q_ref[...]
    k = k_ref[...]
    dk = q.shape[-1]
    s = lax.dot_general(q, k, (((1,), (1,)), ((), ())),
                        preferred_element_type=jnp.float32)
    s = s / jnp.sqrt(jnp.float32(dk))
    m = jnp.max(s, axis=1, keepdims=True)
    num = jnp.exp(s - m)
    den = jnp.sum(num, axis=1, keepdims=True)
    p = num / den
    n = p.shape[1]
    colf = lax.broadcasted_iota(jnp.int32, p.shape, 1).astype(jnp.float32)
    ws, js = [], []
    work = p
    for _ in range(_TOPK):
        mw = jnp.max(work, axis=1, keepdims=True)
        cand = jnp.where(work == mw, colf, jnp.float32(n))
        jf = jnp.min(cand, axis=1, keepdims=True)
        ws.append(mw)
        js.append(jf)
        work = jnp.where(colf == jf, -1.0, work)
    w_ref[...] = jnp.concatenate(ws, axis=1)
    i_ref[...] = jnp.concatenate(js, axis=1).astype(jnp.int32)


def _topk_tc(query, keys):
    nq, dk = query.shape
    n = keys.shape[0]
    grid = nq // _ROWS
    return pl.pallas_call(
        _topk_body,
        grid=(grid,),
        in_specs=[
            pl.BlockSpec((_ROWS, dk), lambda i: (i, 0)),
            pl.BlockSpec((n, dk), lambda i: (0, 0)),
        ],
        out_specs=[
            pl.BlockSpec((_ROWS, _TOPK), lambda i: (i, 0)),
            pl.BlockSpec((_ROWS, _TOPK), lambda i: (i, 0)),
        ],
        out_shape=[
            jax.ShapeDtypeStruct((nq, _TOPK), jnp.float32),
            jax.ShapeDtypeStruct((nq, _TOPK), jnp.int32),
        ],
    )(query, keys)


def _gather_sc(table, idx, zl, dv):
    v, d = table.shape
    b = idx.shape[0]
    info = plsc.get_sparse_core_info()
    nw = info.num_cores * info.num_subcores
    b_per_w = b // nw
    chunk = 32
    n_chunks = b_per_w // chunk
    mesh = plsc.VectorSubcoreMesh(core_axis_name="c", subcore_axis_name="s")

    @functools.partial(
        pl.kernel, mesh=mesh,
        out_type=jax.ShapeDtypeStruct((b, d), jnp.float32),
        scratch_types=[
            pltpu.VMEM((b_per_w,), jnp.int32),
            pltpu.VMEM((2, chunk, d), jnp.float32),
            pltpu.SemaphoreType.DMA,
            pltpu.SemaphoreType.DMA,
        ],
    )
    def k(table_hbm, idx_hbm, out_hbm, idx_v, rows_v, gsem, ssem):
        wid = lax.axis_index("s") * info.num_cores + lax.axis_index("c")
        base = wid * b_per_w
        pltpu.sync_copy(idx_hbm.at[pl.ds(base, b_per_w)], idx_v)

        def gather_start(c, buf):
            return pltpu.async_copy(
                table_hbm.at[idx_v.at[pl.ds(c * chunk, chunk)]],
                rows_v.at[buf], gsem)

        def scatter_start(c, buf):
            return pltpu.async_copy(
                rows_v.at[buf], out_hbm.at[pl.ds(base + c * chunk, chunk)],
                ssem)

        def halfstep(c, buf):
            s = scatter_start(c, buf)
            gather_start(c + 1, 1 - buf).wait()
            s.wait()

        gather_start(0, 0).wait()

        def body(c, carry):
            halfstep(2 * c, 0)
            halfstep(2 * c + 1, 1)
            return carry

        lax.fori_loop(0, (n_chunks - 1) // 2, body, 0, unroll=False)
        if n_chunks % 2 == 0:
            halfstep(n_chunks - 2, 0)
        scatter_start(n_chunks - 1, (n_chunks - 1) % 2).wait()

    return k(table, idx)


def kernel(query, keys, values):
    v, l, dv = values.shape
    nq = query.shape[0]
    nchunk = 4
    qc = nq // nchunk
    table = values.reshape(v, l * dv)
    ws, outs = [], []
    for c in range(nchunk):
        wc, ic = _topk_tc(lax.slice_in_dim(query, c * qc, (c + 1) * qc), keys)
        oc = _gather_sc(table, ic.reshape(-1), l, dv)
        ws.append(wc.reshape(-1))
        outs.append(oc.reshape(1, -1, l, dv))
    out = jnp.concatenate(outs, axis=0).reshape(-1, l, dv)
    return jnp.concatenate(ws), out


# speedup vs baseline: 1.3532x; 1.3532x over previous
import functools

import jax
import jax.numpy as jnp
from jax import lax
from jax.experimental import pallas as pl
from jax.experimental.pallas import tpu as pltpu
from jax.experimental.pallas import tpu_sc as plsc

_TOPK = 8
_ROWS = 256


def _topk_body(q_ref, k_ref, w_ref, i_ref):
    q = q_ref[...]
    k = k_ref[...]
    dk = q.shape[-1]
    s = lax.dot_general(q, k, (((1,), (1,)), ((), ())),
                        preferred_element_type=jnp.float32)
    s = s / jnp.sqrt(jnp.float32(dk))
    m = jnp.max(s, axis=1, keepdims=True)
    num = jnp.exp(s - m)
    den = jnp.sum(num, axis=1, keepdims=True)
    p = num / den
    n = p.shape[1]
    colf = lax.broadcasted_iota(jnp.int32, p.shape, 1).astype(jnp.float32)
    ws, js = [], []
    work = p
    for _ in range(_TOPK):
        mw = jnp.max(work, axis=1, keepdims=True)
        cand = jnp.where(work == mw, colf, jnp.float32(n))
        jf = jnp.min(cand, axis=1, keepdims=True)
        ws.append(mw)
        js.append(jf)
        work = jnp.where(colf == jf, -1.0, work)
    w_ref[...] = jnp.concatenate(ws, axis=1)
    i_ref[...] = jnp.concatenate(js, axis=1).astype(jnp.int32)


def _topk_tc(query, keys):
    nq, dk = query.shape
    n = keys.shape[0]
    grid = nq // _ROWS
    return pl.pallas_call(
        _topk_body,
        grid=(grid,),
        in_specs=[
            pl.BlockSpec((_ROWS, dk), lambda i: (i, 0)),
            pl.BlockSpec((n, dk), lambda i: (0, 0)),
        ],
        out_specs=[
            pl.BlockSpec((_ROWS, _TOPK), lambda i: (i, 0)),
            pl.BlockSpec((_ROWS, _TOPK), lambda i: (i, 0)),
        ],
        out_shape=[
            jax.ShapeDtypeStruct((nq, _TOPK), jnp.float32),
            jax.ShapeDtypeStruct((nq, _TOPK), jnp.int32),
        ],
    )(query, keys)


def _gather_sc(table, idx, zl, dv):
    v, d = table.shape
    b = idx.shape[0]
    info = plsc.get_sparse_core_info()
    nw = info.num_cores * info.num_subcores
    b_per_w = b // nw
    chunk = 32
    n_chunks = b_per_w // chunk
    mesh = plsc.VectorSubcoreMesh(core_axis_name="c", subcore_axis_name="s")

    @functools.partial(
        pl.kernel, mesh=mesh,
        out_type=jax.ShapeDtypeStruct((b, d), jnp.float32),
        scratch_types=[
            pltpu.VMEM((b_per_w,), jnp.int32),
            pltpu.VMEM((2, chunk, d), jnp.float32),
            pltpu.SemaphoreType.DMA,
            pltpu.SemaphoreType.DMA,
        ],
    )
    def k(table_hbm, idx_hbm, out_hbm, idx_v, rows_v, gsem, ssem):
        wid = lax.axis_index("s") * info.num_cores + lax.axis_index("c")
        base = wid * b_per_w
        pltpu.sync_copy(idx_hbm.at[pl.ds(base, b_per_w)], idx_v)

        def gather_start(c, buf):
            return pltpu.async_copy(
                table_hbm.at[idx_v.at[pl.ds(c * chunk, chunk)]],
                rows_v.at[buf], gsem)

        def scatter_start(c, buf):
            return pltpu.async_copy(
                rows_v.at[buf], out_hbm.at[pl.ds(base + c * chunk, chunk)],
                ssem)

        def halfstep(c, buf):
            s = scatter_start(c, buf)
            gather_start(c + 1, 1 - buf).wait()
            s.wait()

        gather_start(0, 0).wait()

        def body(c, carry):
            halfstep(2 * c, 0)
            halfstep(2 * c + 1, 1)
            return carry

        lax.fori_loop(0, (n_chunks - 1) // 2, body, 0, unroll=False)
        if n_chunks % 2 == 0:
            halfstep(n_chunks - 2, 0)
        scatter_start(n_chunks - 1, (n_chunks - 1) % 2).wait()

    return k(table, idx)


def kernel(query, keys, values):
    v, l, dv = values.shape
    w, i = _topk_tc(query, keys)
    out2d = _gather_sc(values.reshape(v, l * dv), i.reshape(-1), l, dv)
    return w.reshape(-1), out2d.reshape(-1, l, dv)
